# baseline (speedup 1.0000x reference)
import functools

import jax
import jax.numpy as jnp
from jax import lax
from jax.experimental import pallas as pl
from jax.experimental.pallas import tpu as pltpu
from jax.experimental.pallas import tpu_sc as plsc

_B, _N, _C = 4, 2048, 1024
_NKEEP = _N // 2
_NP1 = _N + 1
_R = _B * _NP1
_CH = 256
_G = 32
_TILES = 32
_RPT = 256


def _prep_body(ga_row_ref, ga_col_ref, src_ref, w_ref):
    arow = ga_row_ref[0]
    acol = ga_col_ref[0]
    irow = lax.broadcasted_iota(jnp.int32, (1, _N), 1)
    parts = []
    for s in range(_N // _CH):
        a_i = acol[s * _CH:(s + 1) * _CH, :]
        i_i = lax.broadcasted_iota(jnp.int32, (_CH, 1), 0) + s * _CH
        before = (arow > a_i) | ((arow == a_i) & (irow < i_i))
        parts.append(jnp.sum(before.astype(jnp.float32), axis=1, keepdims=True))
    cntcol = jnp.concatenate(parts, axis=0)
    iotacol = lax.broadcasted_iota(jnp.int32, (_N, 1), 0).astype(jnp.float32)
    rankcol = cntcol.astype(jnp.int32)
    src_parts = []
    for c in range(_N // _CH):
        p_i = lax.broadcasted_iota(jnp.int32, (1, _CH), 1) + c * _CH
        onehot = (rankcol == p_i).astype(jnp.float32)
        src_parts.append(jnp.sum(onehot * iotacol, axis=0, keepdims=True))
    src = jnp.concatenate(src_parts, axis=1)
    src_ref[0] = src.astype(jnp.int32)
    wcol = (cntcol >= float(_NKEEP)).astype(jnp.float32) * acol
    w_ref[0] = wcol * (2.0 / jnp.sum(wcol))


def _make_prep(interpret=False):
    return pl.pallas_call(
        _prep_body,
        grid=(_B,),
        in_specs=[
            pl.BlockSpec((1, 1, _N), lambda b: (b, 0, 0)),
            pl.BlockSpec((1, _N, 1), lambda b: (b, 0, 0)),
        ],
        out_specs=[
            pl.BlockSpec((1, 1, _N), lambda b: (b, 0, 0)),
            pl.BlockSpec((1, _N, 1), lambda b: (b, 0, 0)),
        ],
        out_shape=[
            jax.ShapeDtypeStruct((_B, 1, _N), jnp.int32),
            jax.ShapeDtypeStruct((_B, _N, 1), jnp.float32),
        ],
        interpret=interpret,
    )


_MVB = (_NP1 * 32) // 3


def _matvec_body(x_ref, w_ref, out_ref):
    i = pl.program_id(0)
    x3 = x_ref[...].reshape(_MVB // 32, 32, 128)
    w3 = w_ref[...].reshape(_MVB // 32, 32, 1)
    partial = jnp.sum(x3 * w3, axis=0)

    @pl.when(i == 0)
    def _init():
        out_ref[...] = partial

    @pl.when(i > 0)
    def _acc():
        out_ref[...] = out_ref[...] + partial


def _make_matvec(interpret=False):
    return pl.pallas_call(
        _matvec_body,
        grid=(3,),
        in_specs=[
            pl.BlockSpec((_MVB, 128), lambda i: (i, 0)),
            pl.BlockSpec((_MVB, 1), lambda i: (i, 0)),
        ],
        out_specs=pl.BlockSpec((_PR, 128), lambda i: (0, 0)),
        out_shape=jax.ShapeDtypeStruct((_PR, 128), jnp.float32),
        interpret=interpret,
    )


_PR = _B * 8
_TOKCH = 8
_CROWS = _TOKCH * _PR
_NCHUNK = (_NP1 - 1) // _TOKCH
_CPW = _NCHUNK // _TILES


def _sc_body(xp, srcp, tailp, addp, out, xb0, xb1, xb2, idx2d, tidx, apv,
             gs0, gs1, gs2, ss0, ss1, ss2, tsem):
    cid = lax.axis_index("c")
    sid = lax.axis_index("s")
    wid = sid * 2 + cid
    xbufs = [xb0, xb1, xb2]
    gsems = [gs0, gs1, gs2]
    ssems = [ss0, ss1, ss2]

    pltpu.sync_copy(addp, apv)
    pltpu.sync_copy(srcp.at[wid], idx2d)

    def start_gather(j):
        h0 = pltpu.async_copy(
            xp.at[idx2d.at[2 * j]], xbufs[j % 3].at[pl.ds(0, 128)],
            gsems[j % 3])
        h1 = pltpu.async_copy(
            xp.at[idx2d.at[2 * j + 1]], xbufs[j % 3].at[pl.ds(128, 128)],
            gsems[j % 3])
        return (h0, h1)

    def start_store(j):
        r0 = (wid + _TILES * j) * _CROWS
        return pltpu.async_copy(
            xbufs[j % 3], out.at[pl.ds(r0, _CROWS)], ssems[j % 3])

    g = {}
    s = {}
    g[0] = start_gather(0)
    g[1] = start_gather(1)
    for j in range(_CPW):
        if j + 2 < _CPW:
            if j - 1 >= 0:
                s[j - 1].wait()
            g[j + 2] = start_gather(j + 2)
        g[j][0].wait()
        g[j][1].wait()
        xbuf = xbufs[j % 3]
        if j >= 4:
            def q_body(q, c):
                def v_body(v, c2):
                    sl = pl.ds(v * 16, 16)
                    a = apv[q, sl]

                    def t_body(tt, c3):
                        xbuf[q + tt * _PR, sl] = xbuf[q + tt * _PR, sl] + a
                        return c3
                    lax.fori_loop(0, _TOKCH, t_body, 0, unroll=4)
                    return c2
                lax.fori_loop(0, 128 // 16, v_body, 0)
                return c
            lax.fori_loop(0, _PR, q_body, 0)
            if j == 4:
                @pl.when(wid == 0)
                def _fix():
                    def q2(q, c):
                        def v2(v, c2):
                            sl = pl.ds(v * 16, 16)
                            xbuf[q, sl] = xbuf[q, sl] - apv[q, sl]
                            return c2
                        lax.fori_loop(0, 128 // 16, v2, 0)
                        return c
                    lax.fori_loop(0, _PR, q2, 0)
        s[j] = start_store(j)
    for j in range(_CPW - 3, _CPW):
        s[j].wait()

    @pl.when(wid == _TILES - 1)
    def _tail():
        pltpu.sync_copy(tailp, tidx)
        pltpu.async_copy(
            xp.at[tidx.at[0]], xbufs[0].at[pl.ds(0, _PR)], tsem).wait()

        def q_body(q, c):
            def v_body(v, c2):
                sl = pl.ds(v * 16, 16)
                xbufs[0][q, sl] = xbufs[0][q, sl] + apv[q, sl]
                return c2
            lax.fori_loop(0, 128 // 16, v_body, 0)
            return c
        lax.fori_loop(0, _PR, q_body, 0)
        pltpu.sync_copy(xbufs[0].at[pl.ds(0, _PR)],
                        out.at[pl.ds(_NCHUNK * _CROWS, _PR)])


@functools.cache
def _make_sc_gather():
    return functools.partial(
        pl.kernel,
        out_type=jax.ShapeDtypeStruct((_NP1 * _PR, 128), jnp.float32),
        mesh=plsc.VectorSubcoreMesh(core_axis_name="c", subcore_axis_name="s"),
        scratch_types=[
            pltpu.VMEM((_CROWS, 128), jnp.float32),
            pltpu.VMEM((_CROWS, 128), jnp.float32),
            pltpu.VMEM((_CROWS, 128), jnp.float32),
            pltpu.VMEM((16, 128), jnp.int32),
            pltpu.VMEM((1, _PR), jnp.int32),
            pltpu.VMEM((_PR, 128), jnp.float32),
            pltpu.SemaphoreType.DMA,
            pltpu.SemaphoreType.DMA,
            pltpu.SemaphoreType.DMA,
            pltpu.SemaphoreType.DMA,
            pltpu.SemaphoreType.DMA,
            pltpu.SemaphoreType.DMA,
            pltpu.SemaphoreType.DMA,
        ],
    )(_sc_body)


def kernel(x_, global_attn, ori_indices):
    del ori_indices
    src, w = _make_prep()(
        global_attn.reshape(_B, 1, _N),
        global_attn.reshape(_B, _N, 1),
    )
    src_full = jnp.concatenate(
        [jnp.zeros((_B, 1), jnp.int32), src.reshape(_B, _N) + 1],
        axis=1)
    piece = (src_full.transpose(1, 0)[:, None, :] * _PR
             + (jnp.arange(8, dtype=jnp.int32) * _B)[None, :, None]
             + jnp.arange(_B, dtype=jnp.int32)[None, None, :])
    pflat = piece.reshape(_NP1 * _PR)
    srcp = (pflat[:_NCHUNK * _CROWS].reshape(_CPW, _TILES, _CROWS)
            .transpose(1, 0, 2).reshape(_TILES, 2 * _CPW, 128))
    tailp = pflat[_NCHUNK * _CROWS:].reshape(1, _PR)
    xp = (x_.reshape(_B, _NP1, 8, 128).transpose(1, 2, 0, 3)
          .reshape(_NP1 * _PR, 128))
    w_full = jnp.concatenate(
        [jnp.zeros((_B, 1), jnp.float32), w.reshape(_B, _N)],
        axis=1)
    wp = jnp.broadcast_to(
        w_full.transpose(1, 0)[:, None, :], (_NP1, 8, _B)
    ).reshape(_NP1 * _PR, 1)
    addp = _make_matvec()(xp, wp)
    out2 = _make_sc_gather()(xp, srcp, tailp, addp)
    return (out2.reshape(_NP1, 8, _B, 128).transpose(2, 0, 1, 3)
            .reshape(_B, _NP1, _C))


# speedup vs baseline: 1.1295x; 1.1295x over previous
import functools

import jax
import jax.numpy as jnp
from jax import lax
from jax.experimental import pallas as pl
from jax.experimental.pallas import tpu as pltpu
from jax.experimental.pallas import tpu_sc as plsc

_B, _N, _C = 4, 2048, 1024
_NKEEP = _N // 2
_NP1 = _N + 1
_R = _B * _NP1
_CH = 256
_G = 32
_TILES = 32
_RPT = 256


def _prep_body(ga_row_ref, ga_col_ref, x_ref, src_ref, add2_ref):
    arow = ga_row_ref[0]
    acol = ga_col_ref[0]
    irow = lax.broadcasted_iota(jnp.int32, (1, _N), 1)
    parts = []
    for s in range(_N // _CH):
        a_i = acol[s * _CH:(s + 1) * _CH, :]
        i_i = lax.broadcasted_iota(jnp.int32, (_CH, 1), 0) + s * _CH
        before = (arow > a_i) | ((arow == a_i) & (irow < i_i))
        parts.append(jnp.sum(before.astype(jnp.float32), axis=1, keepdims=True))
    cntcol = jnp.concatenate(parts, axis=0)
    iotacol = lax.broadcasted_iota(jnp.int32, (_N, 1), 0).astype(jnp.float32)
    rankcol = cntcol.astype(jnp.int32)
    src_parts = []
    for c in range(_N // _CH):
        p_i = lax.broadcasted_iota(jnp.int32, (1, _CH), 1) + c * _CH
        onehot = (rankcol == p_i).astype(jnp.float32)
        src_parts.append(jnp.sum(onehot * iotacol, axis=0, keepdims=True))
    src = jnp.concatenate(src_parts, axis=1)
    src_ref[0] = src.astype(jnp.int32)
    wcol = (cntcol >= float(_NKEEP)).astype(jnp.float32) * acol
    x = x_ref[0, 1:, :]
    t = lax.dot_general(wcol, x, (((0,), (0,)), ((), ())),
                        preferred_element_type=jnp.float32,
                        precision=lax.Precision.HIGHEST)
    add2_ref[0] = t * (2.0 / jnp.sum(wcol))


def _make_prep(interpret=False):
    return pl.pallas_call(
        _prep_body,
        grid=(_B,),
        in_specs=[
            pl.BlockSpec((1, 1, _N), lambda b: (b, 0, 0)),
            pl.BlockSpec((1, _N, 1), lambda b: (b, 0, 0)),
            pl.BlockSpec((1, _NP1, _C), lambda b: (b, 0, 0)),
        ],
        out_specs=[
            pl.BlockSpec((1, 1, _N), lambda b: (b, 0, 0)),
            pl.BlockSpec((1, 1, _C), lambda b: (b, 0, 0)),
        ],
        out_shape=[
            jax.ShapeDtypeStruct((_B, 1, _N), jnp.int32),
            jax.ShapeDtypeStruct((_B, 1, _C), jnp.float32),
        ],
        interpret=interpret,
    )


_PR = _B * 8
_TOKCH = 8
_CROWS = _TOKCH * _PR
_NCHUNK = (_NP1 - 1) // _TOKCH
_CPW = _NCHUNK // _TILES


def _sc_body(xp, srcp, tailp, addp, out, xb0, xb1, xb2, idx2d, tidx, apv,
             gs0, gs1, gs2, ss0, ss1, ss2, tsem):
    cid = lax.axis_index("c")
    sid = lax.axis_index("s")
    wid = sid * 2 + cid
    xbufs = [xb0, xb1, xb2]
    gsems = [gs0, gs1, gs2]
    ssems = [ss0, ss1, ss2]

    pltpu.sync_copy(addp, apv)
    pltpu.sync_copy(srcp.at[wid], idx2d)

    def start_gather(j):
        h0 = pltpu.async_copy(
            xp.at[idx2d.at[2 * j]], xbufs[j % 3].at[pl.ds(0, 128)],
            gsems[j % 3])
        h1 = pltpu.async_copy(
            xp.at[idx2d.at[2 * j + 1]], xbufs[j % 3].at[pl.ds(128, 128)],
            gsems[j % 3])
        return (h0, h1)

    def start_store(j):
        r0 = (wid + _TILES * j) * _CROWS
        return pltpu.async_copy(
            xbufs[j % 3], out.at[pl.ds(r0, _CROWS)], ssems[j % 3])

    g = {}
    s = {}
    g[0] = start_gather(0)
    g[1] = start_gather(1)
    for j in range(_CPW):
        if j + 2 < _CPW:
            if j - 1 >= 0:
                s[j - 1].wait()
            g[j + 2] = start_gather(j + 2)
        g[j][0].wait()
        g[j][1].wait()
        xbuf = xbufs[j % 3]
        if j >= 4:
            def q_body(q, c):
                def v_body(v, c2):
                    sl = pl.ds(v * 16, 16)
                    a = apv[q, sl]

                    def t_body(tt, c3):
                        xbuf[q + tt * _PR, sl] = xbuf[q + tt * _PR, sl] + a
                        return c3
                    lax.fori_loop(0, _TOKCH, t_body, 0, unroll=4)
                    return c2
                lax.fori_loop(0, 128 // 16, v_body, 0)
                return c
            lax.fori_loop(0, _PR, q_body, 0)
            if j == 4:
                @pl.when(wid == 0)
                def _fix():
                    def q2(q, c):
                        def v2(v, c2):
                            sl = pl.ds(v * 16, 16)
                            xbuf[q, sl] = xbuf[q, sl] - apv[q, sl]
                            return c2
                        lax.fori_loop(0, 128 // 16, v2, 0)
                        return c
                    lax.fori_loop(0, _PR, q2, 0)
        s[j] = start_store(j)
    for j in range(_CPW - 3, _CPW):
        s[j].wait()

    @pl.when(wid == _TILES - 1)
    def _tail():
        pltpu.sync_copy(tailp, tidx)
        pltpu.async_copy(
            xp.at[tidx.at[0]], xbufs[0].at[pl.ds(0, _PR)], tsem).wait()

        def q_body(q, c):
            def v_body(v, c2):
                sl = pl.ds(v * 16, 16)
                xbufs[0][q, sl] = xbufs[0][q, sl] + apv[q, sl]
                return c2
            lax.fori_loop(0, 128 // 16, v_body, 0)
            return c
        lax.fori_loop(0, _PR, q_body, 0)
        pltpu.sync_copy(xbufs[0].at[pl.ds(0, _PR)],
                        out.at[pl.ds(_NCHUNK * _CROWS, _PR)])


@functools.cache
def _make_sc_gather():
    return functools.partial(
        pl.kernel,
        out_type=jax.ShapeDtypeStruct((_NP1 * _PR, 128), jnp.float32),
        mesh=plsc.VectorSubcoreMesh(core_axis_name="c", subcore_axis_name="s"),
        scratch_types=[
            pltpu.VMEM((_CROWS, 128), jnp.float32),
            pltpu.VMEM((_CROWS, 128), jnp.float32),
            pltpu.VMEM((_CROWS, 128), jnp.float32),
            pltpu.VMEM((16, 128), jnp.int32),
            pltpu.VMEM((1, _PR), jnp.int32),
            pltpu.VMEM((_PR, 128), jnp.float32),
            pltpu.SemaphoreType.DMA,
            pltpu.SemaphoreType.DMA,
            pltpu.SemaphoreType.DMA,
            pltpu.SemaphoreType.DMA,
            pltpu.SemaphoreType.DMA,
            pltpu.SemaphoreType.DMA,
            pltpu.SemaphoreType.DMA,
        ],
    )(_sc_body)


def kernel(x_, global_attn, ori_indices):
    del ori_indices
    src, add2 = _make_prep()(
        global_attn.reshape(_B, 1, _N),
        global_attn.reshape(_B, _N, 1),
        x_,
    )
    src_full = jnp.concatenate(
        [jnp.zeros((_B, 1), jnp.int32), src.reshape(_B, _N) + 1],
        axis=1)
    piece = (src_full.transpose(1, 0)[:, None, :] * _PR
             + (jnp.arange(8, dtype=jnp.int32) * _B)[None, :, None]
             + jnp.arange(_B, dtype=jnp.int32)[None, None, :])
    pflat = piece.reshape(_NP1 * _PR)
    srcp = (pflat[:_NCHUNK * _CROWS].reshape(_CPW, _TILES, _CROWS)
            .transpose(1, 0, 2).reshape(_TILES, 2 * _CPW, 128))
    tailp = pflat[_NCHUNK * _CROWS:].reshape(1, _PR)
    addp = (add2.reshape(_B, 8, 128).transpose(1, 0, 2)
            .reshape(_PR, 128))
    xp = (x_.reshape(_B, _NP1, 8, 128).transpose(1, 2, 0, 3)
          .reshape(_NP1 * _PR, 128))
    out2 = _make_sc_gather()(xp, srcp, tailp, addp)
    return (out2.reshape(_NP1, 8, _B, 128).transpose(2, 0, 1, 3)
            .reshape(_B, _NP1, _C))
